# baseline (speedup 1.0000x reference)
import functools

import jax
import jax.numpy as jnp
from jax import lax
from jax.experimental import pallas as pl
from jax.experimental.pallas import tpu as pltpu
from jax.experimental.pallas import tpu_sc as plsc


def kernel(tokens, table):
    B0, S = tokens.shape
    V, D = table.shape
    B = B0 * S
    info = plsc.get_sparse_core_info()
    NC, NS = info.num_cores, info.num_subcores
    NW = NC * NS
    C = 128
    NBUF = 4
    bw = B // NW
    nch = bw // C
    R = nch // NBUF

    NT = V // 128
    TAIL = V - NT * 128
    TPW = -(-NT // NW)
    TPW += TPW % 2

    idx = tokens.reshape(NW, nch, C).astype(jnp.int32)
    mesh = plsc.VectorSubcoreMesh(core_axis_name="c", subcore_axis_name="s")
    params = pltpu.CompilerParams(
        use_tc_tiling_on_sc=True, needs_layout_passes=False)

    @functools.partial(
        pl.kernel,
        mesh=mesh,
        compiler_params=params,
        out_type=jax.ShapeDtypeStruct((V, 128), jnp.float32),
        scratch_types=(
            [pltpu.VMEM((64, 128), jnp.float32) for _ in range(2)]
            + [pltpu.VMEM((128, 128), jnp.float32) for _ in range(2)]
            + [pltpu.VMEM((64, 64), jnp.float32)]
            + [pltpu.SemaphoreType.DMA for _ in range(4)]
        ),
    )
    def trans(tt_hbm, tg_hbm, vin0, vin1, vout0, vout1, vtail, *sems):
        vin = (vin0, vin1)
        vout = (vout0, vout1)
        isem = sems[:2]
        wsem = sems[2:]
        wid = lax.axis_index("s") * NC + lax.axis_index("c")
        iota = lax.iota(jnp.int32, 16)

        def blk(i):
            return jnp.minimum(wid + NW * i, NT - 1) * 128

        def rd(i, b):
            return pltpu.make_async_copy(
                tt_hbm.at[:, pl.ds(blk(i), 128)], vin[b], isem[b])

        def wr(i, b):
            return pltpu.make_async_copy(
                vout[b], tg_hbm.at[pl.ds(blk(i), 128)], wsem[b])

        def transpose(src, dst):
            def row(r, carry):
                cr = jnp.full((16,), r, jnp.int32)
                for f0 in range(0, 64, 16):
                    vals = plsc.load_gather(src, [f0 + iota, cr])
                    dst[r, pl.ds(f0, 16)] = vals
                return carry
            lax.fori_loop(0, 128, row, 0)

        rd(0, 0).start()
        rd(1, 1).start()
        for b in range(2):
            rd(b, b).wait()
            transpose(vin[b], vout[b])
            wr(b, b).start()
            rd(2 + b, b).start()

        def body(i2, carry):
            for b in range(2):
                i = 2 * i2 + b
                rd(i, b).wait()
                wr(i - 2, b).wait()
                transpose(vin[b], vout[b])
                wr(i, b).start()
                rd(i + 2, b).start()
            return carry

        lax.fori_loop(1, TPW // 2, body, 0)

        for b in range(2):
            rd(TPW + b, b).wait()
            wr(TPW - 2 + b, b).wait()

        @pl.when(wid == 0)
        def _tail():
            pltpu.sync_copy(tt_hbm.at[:, pl.ds(NT * 128, TAIL)], vtail)

            def row(r, carry):
                cr = jnp.full((16,), r, jnp.int32)
                for f0 in range(0, 64, 16):
                    vals = plsc.load_gather(vtail, [f0 + iota, cr])
                    vout0[r, pl.ds(f0, 16)] = vals
                return carry
            lax.fori_loop(0, TAIL, row, 0)
            pltpu.sync_copy(
                vout0.at[pl.ds(0, TAIL)], tg_hbm.at[pl.ds(NT * 128, TAIL)])

    @functools.partial(
        pl.kernel,
        mesh=mesh,
        compiler_params=params,
        out_type=jax.ShapeDtypeStruct((B, 128), jnp.float32),
        scratch_types=(
            [pltpu.VMEM((nch, C), jnp.int32)]
            + [pltpu.VMEM((C, 128), jnp.float32) for _ in range(NBUF)]
            + [pltpu.SemaphoreType.DMA for _ in range(2 * NBUF)]
        ),
    )
    def emb(idx_hbm, table_hbm, out_hbm, idx_v, *rest):
        bufs = rest[:NBUF]
        gsem = rest[NBUF:2 * NBUF]
        wsem = rest[2 * NBUF:]
        wid = lax.axis_index("s") * NC + lax.axis_index("c")
        base = wid * bw
        pltpu.sync_copy(idx_hbm.at[wid], idx_v)

        def gather(j, b):
            return pltpu.make_async_copy(
                table_hbm.at[idx_v.at[j]], bufs[b], gsem[b])

        def write(j, b):
            return pltpu.make_async_copy(
                bufs[b], out_hbm.at[pl.ds(base + j * C, C)], wsem[b])

        for b in range(NBUF):
            gather(b, b).start()

        def body(r, carry):
            jp = (r - 1) * NBUF
            j0 = r * NBUF
            for b in range(NBUF):
                gather(jp + b, b).wait()
                write(jp + b, b).start()
            for b in range(NBUF):
                write(jp + b, b).wait()
                gather(j0 + b, b).start()
            return carry

        lax.fori_loop(1, R, body, 0)

        jl = (R - 1) * NBUF
        for b in range(NBUF):
            gather(jl + b, b).wait()
            write(jl + b, b).start()
        for b in range(NBUF):
            write(jl + b, b).wait()

    tableG = trans(table.T)
    out = emb(idx, tableG)
    return out[:, :D].reshape(B0, S, D)
